# Initial kernel scaffold; baseline (speedup 1.0000x reference)
import jax
import jax.numpy as jnp
from jax.experimental import pallas as pl


def kernel(score):
    raise NotImplementedError("write your pallas kernel here")


# speedup vs baseline: 3.8601x; 3.8601x over previous
import jax
import jax.numpy as jnp
from jax import lax
from jax.experimental import pallas as pl
from jax.experimental.pallas import tpu as pltpu
from jax.experimental.pallas import tpu_sc as plsc

N = 32768
P = 64
L = 16
NC = 2
NS = 16
NW = NC * NS
ROWS_W = N // NW
CHUNK_ROWS = 128
NCHUNK = ROWS_W // CHUNK_ROWS
GROUPS = CHUNK_ROWS // L
CHUNK_WORDS = CHUNK_ROWS * P


def _topk_mask_body(score_hbm, out_hbm, in0, in1, ot0, ot1, si0, si1, so0, so1):
    cid = lax.axis_index("c")
    sid = lax.axis_index("s")
    wid = sid * NC + cid
    base_word = wid * (ROWS_W * P)

    ins = (in0, in1)
    outs = (ot0, ot1)
    isems = (si0, si1)
    osems = (so0, so1)

    lane = lax.iota(jnp.int32, L)
    zeros_i = jnp.zeros((L,), jnp.int32)
    ones_i = jnp.ones((L,), jnp.int32)
    ninf = jnp.full((L,), float("-inf"), jnp.float32)

    def in_copy(k, b):
        return pltpu.make_async_copy(
            score_hbm.at[pl.ds(base_word + k * CHUNK_WORDS, CHUNK_WORDS)],
            ins[b], isems[b])

    def out_copy(k, b):
        return pltpu.make_async_copy(
            outs[b],
            out_hbm.at[pl.ds(base_word + k * CHUNK_WORDS, CHUNK_WORDS)],
            osems[b])

    in_copy(0, 0).start()

    def pair_body(p, carry):
        for b in range(2):
            k = p * 2 + b
            in_copy(k, b).wait()

            @pl.when(k + 1 < NCHUNK)
            def _():
                in_copy(k + 1, 1 - b).start()

            @pl.when(k >= 2)
            def _():
                out_copy(k - 2, b).wait()

            def group_body(g, gcarry):
                gbase = lane * P + g * (L * P)
                m1 = ninf
                m2 = ninf
                a1 = zeros_i
                a2 = zeros_i
                for c in range(P):
                    idxv = gbase + c
                    v = plsc.load_gather(ins[b], [idxv])
                    gt1 = v > m1
                    u = jnp.minimum(m1, v)
                    ui = jnp.where(gt1, a1, idxv)
                    gt2 = gt1 | (u > m2)
                    m1 = jnp.maximum(m1, v)
                    a1 = jnp.where(gt1, idxv, a1)
                    m2 = jnp.maximum(m2, u)
                    a2 = jnp.where(gt2, ui, a2)
                for z in range(P):
                    outs[b][pl.ds(g * (L * P) + z * L, L)] = zeros_i
                plsc.store_scatter(outs[b], [a1], ones_i)
                plsc.store_scatter(outs[b], [a2], ones_i)
                return gcarry

            lax.fori_loop(0, GROUPS, group_body, 0)
            out_copy(k, b).start()
        return carry

    lax.fori_loop(0, NCHUNK // 2, pair_body, 0)
    out_copy(NCHUNK - 2, 0).wait()
    out_copy(NCHUNK - 1, 1).wait()


def kernel(score):
    flat = score.reshape(-1)
    mesh = plsc.VectorSubcoreMesh(
        core_axis_name="c", subcore_axis_name="s",
        num_cores=NC, num_subcores=NS)
    out = pl.kernel(
        _topk_mask_body,
        out_type=jax.ShapeDtypeStruct((N * P,), jnp.int32),
        mesh=mesh,
        compiler_params=pltpu.CompilerParams(needs_layout_passes=False),
        scratch_types=[
            pltpu.VMEM((CHUNK_WORDS,), jnp.float32),
            pltpu.VMEM((CHUNK_WORDS,), jnp.float32),
            pltpu.VMEM((CHUNK_WORDS,), jnp.int32),
            pltpu.VMEM((CHUNK_WORDS,), jnp.int32),
            pltpu.SemaphoreType.DMA,
            pltpu.SemaphoreType.DMA,
            pltpu.SemaphoreType.DMA,
            pltpu.SemaphoreType.DMA,
        ],
    )(flat)
    return out.reshape(N, P)


# speedup vs baseline: 4.5239x; 1.1720x over previous
import jax
import jax.numpy as jnp
from jax import lax
from jax.experimental import pallas as pl
from jax.experimental.pallas import tpu as pltpu
from jax.experimental.pallas import tpu_sc as plsc

N = 32768
P = 64
L = 16
NC = 2
NS = 16
NW = NC * NS
ROWS_W = N // NW
CHUNK_ROWS = 128
NCHUNK = ROWS_W // CHUNK_ROWS
GROUPS = CHUNK_ROWS // L
CHUNK_WORDS = CHUNK_ROWS * P


def _topk_mask_body(score_hbm, out_hbm, in0, in1, ot0, ot1, si0, si1, so0, so1):
    cid = lax.axis_index("c")
    sid = lax.axis_index("s")
    wid = sid * NC + cid
    base_word = wid * (ROWS_W * P)

    ins = (in0, in1)
    outs = (ot0, ot1)
    isems = (si0, si1)
    osems = (so0, so1)

    lane = lax.iota(jnp.int32, L)
    zeros_i = jnp.zeros((L,), jnp.int32)
    ones_i = jnp.ones((L,), jnp.int32)
    ninf = jnp.full((L,), float("-inf"), jnp.float32)

    def in_copy(k, b):
        return pltpu.make_async_copy(
            score_hbm.at[pl.ds(base_word + k * CHUNK_WORDS, CHUNK_WORDS)],
            ins[b], isems[b])

    def out_copy(k, b):
        return pltpu.make_async_copy(
            outs[b],
            out_hbm.at[pl.ds(base_word + k * CHUNK_WORDS, CHUNK_WORDS)],
            osems[b])

    in_copy(0, 0).start()

    def pair_body(p, carry):
        for b in range(2):
            k = p * 2 + b
            in_copy(k, b).wait()

            @pl.when(k + 1 < NCHUNK)
            def _():
                in_copy(k + 1, 1 - b).start()

            @pl.when(k >= 2)
            def _():
                out_copy(k - 2, b).wait()

            def group_body(g, gcarry):
                gbase = lane * P + g * (L * P)
                m1 = ninf
                m2 = ninf
                a1 = zeros_i
                a2 = zeros_i
                for c in range(P):
                    rc = (lane + c) & (P - 1)
                    idxv = gbase + rc
                    v = plsc.load_gather(ins[b], [idxv])
                    gt1 = (v > m1) | ((v == m1) & (idxv < a1))
                    u = jnp.where(gt1, m1, v)
                    ui = jnp.where(gt1, a1, idxv)
                    m1 = jnp.where(gt1, v, m1)
                    a1 = jnp.where(gt1, idxv, a1)
                    gt2 = (u > m2) | ((u == m2) & (ui < a2))
                    m2 = jnp.where(gt2, u, m2)
                    a2 = jnp.where(gt2, ui, a2)
                for z in range(P):
                    outs[b][pl.ds(g * (L * P) + z * L, L)] = zeros_i
                plsc.store_scatter(outs[b], [a1], ones_i)
                plsc.store_scatter(outs[b], [a2], ones_i)
                return gcarry

            lax.fori_loop(0, GROUPS, group_body, 0)
            out_copy(k, b).start()
        return carry

    lax.fori_loop(0, NCHUNK // 2, pair_body, 0)
    out_copy(NCHUNK - 2, 0).wait()
    out_copy(NCHUNK - 1, 1).wait()


def kernel(score):
    flat = score.reshape(-1)
    mesh = plsc.VectorSubcoreMesh(
        core_axis_name="c", subcore_axis_name="s",
        num_cores=NC, num_subcores=NS)
    out = pl.kernel(
        _topk_mask_body,
        out_type=jax.ShapeDtypeStruct((N * P,), jnp.int32),
        mesh=mesh,
        compiler_params=pltpu.CompilerParams(needs_layout_passes=False),
        scratch_types=[
            pltpu.VMEM((CHUNK_WORDS,), jnp.float32),
            pltpu.VMEM((CHUNK_WORDS,), jnp.float32),
            pltpu.VMEM((CHUNK_WORDS,), jnp.int32),
            pltpu.VMEM((CHUNK_WORDS,), jnp.int32),
            pltpu.SemaphoreType.DMA,
            pltpu.SemaphoreType.DMA,
            pltpu.SemaphoreType.DMA,
            pltpu.SemaphoreType.DMA,
        ],
    )(flat)
    return out.reshape(N, P)


# speedup vs baseline: 6.6432x; 1.4685x over previous
import jax
import jax.numpy as jnp
from jax import lax
from jax.experimental import pallas as pl
from jax.experimental.pallas import tpu as pltpu
from jax.experimental.pallas import tpu_sc as plsc

N = 32768
P = 64
L = 16
NC = 2
NS = 16
NW = NC * NS
ROWS_W = N // NW
CHUNK_ROWS = 128
NCHUNK = ROWS_W // CHUNK_ROWS
GROUPS = CHUNK_ROWS // L


def _topk_mask_body(score_hbm, out_hbm, in0, in1, ot0, ot1, si0, si1, so0, so1):
    cid = lax.axis_index("c")
    sid = lax.axis_index("s")
    wid = sid * NC + cid
    base_row = wid * ROWS_W

    ins = (in0, in1)
    outs = (ot0, ot1)
    isems = (si0, si1)
    osems = (so0, so1)

    lane = lax.iota(jnp.int32, L)
    zeros_i = jnp.zeros((L,), jnp.int32)
    ones_i = jnp.ones((L,), jnp.int32)
    ninf = jnp.full((L,), float("-inf"), jnp.float32)

    def in_copy(k, b):
        return pltpu.make_async_copy(
            score_hbm.at[pl.ds(base_row + k * CHUNK_ROWS, CHUNK_ROWS)],
            ins[b], isems[b])

    def out_copy(k, b):
        return pltpu.make_async_copy(
            outs[b],
            out_hbm.at[pl.ds(base_row + k * CHUNK_ROWS, CHUNK_ROWS)],
            osems[b])

    in_copy(0, 0).start()

    def pair_body(p, carry):
        for b in range(2):
            k = p * 2 + b
            in_copy(k, b).wait()

            @pl.when(k + 1 < NCHUNK)
            def _():
                in_copy(k + 1, 1 - b).start()

            @pl.when(k >= 2)
            def _():
                out_copy(k - 2, b).wait()

            def group_body(g, gcarry):
                rowv = lane + g * L
                m1 = ninf
                m2 = ninf
                a1 = zeros_i
                a2 = zeros_i
                for c in range(P):
                    rc = (lane + c) & (P - 1)
                    v = plsc.load_gather(ins[b], [rowv, rc])
                    gt1 = (v > m1) | ((v == m1) & (rc < a1))
                    u = jnp.where(gt1, m1, v)
                    ui = jnp.where(gt1, a1, rc)
                    m1 = jnp.where(gt1, v, m1)
                    a1 = jnp.where(gt1, rc, a1)
                    gt2 = (u > m2) | ((u == m2) & (ui < a2))
                    m2 = jnp.where(gt2, u, m2)
                    a2 = jnp.where(gt2, ui, a2)
                for r in range(L):
                    for q in range(P // L):
                        outs[b][g * L + r, pl.ds(q * L, L)] = zeros_i
                plsc.store_scatter(outs[b], [rowv, a1], ones_i)
                plsc.store_scatter(outs[b], [rowv, a2], ones_i)
                return gcarry

            lax.fori_loop(0, GROUPS, group_body, 0)
            out_copy(k, b).start()
        return carry

    lax.fori_loop(0, NCHUNK // 2, pair_body, 0)
    out_copy(NCHUNK - 2, 0).wait()
    out_copy(NCHUNK - 1, 1).wait()


def kernel(score):
    mesh = plsc.VectorSubcoreMesh(
        core_axis_name="c", subcore_axis_name="s",
        num_cores=NC, num_subcores=NS)
    out = pl.kernel(
        _topk_mask_body,
        out_type=jax.ShapeDtypeStruct((N, P), jnp.int32),
        mesh=mesh,
        compiler_params=pltpu.CompilerParams(needs_layout_passes=False),
        scratch_types=[
            pltpu.VMEM((CHUNK_ROWS, P), jnp.float32),
            pltpu.VMEM((CHUNK_ROWS, P), jnp.float32),
            pltpu.VMEM((CHUNK_ROWS, P), jnp.int32),
            pltpu.VMEM((CHUNK_ROWS, P), jnp.int32),
            pltpu.SemaphoreType.DMA,
            pltpu.SemaphoreType.DMA,
            pltpu.SemaphoreType.DMA,
            pltpu.SemaphoreType.DMA,
        ],
    )(score)
    return out


# speedup vs baseline: 13.1692x; 1.9824x over previous
import jax
import jax.numpy as jnp
from jax import lax
from jax.experimental import pallas as pl
from jax.experimental.pallas import tpu as pltpu
from jax.experimental.pallas import tpu_sc as plsc

N = 32768
P = 64
L = 16
NC = 2
NS = 16
NW = NC * NS
TOK_W = N // NW
TB = 128
NCHUNK = TOK_W // TB
GROUPS = TB // L


def _topk_mask_body(in4, out4, in0, in1, ot0, ot1, si0, si1, so0, so1):
    cid = lax.axis_index("c")
    sid = lax.axis_index("s")
    wid = sid * NC + cid
    jbase = wid * NCHUNK

    ins = (in0, in1)
    outs = (ot0, ot1)
    isems = (si0, si1)
    osems = (so0, so1)

    lane = lax.iota(jnp.int32, L)
    zeros_i = jnp.zeros((L,), jnp.int32)
    ones_i = jnp.ones((L,), jnp.int32)
    ninf = jnp.full((L,), float("-inf"), jnp.float32)

    def in_copy(k, b):
        return pltpu.make_async_copy(in4.at[:, jbase + k], ins[b], isems[b])

    def out_copy(k, b):
        return pltpu.make_async_copy(outs[b], out4.at[:, jbase + k], osems[b])

    in_copy(0, 0).start()

    def pair_body(pp, carry):
        for b in range(2):
            k = pp * 2 + b
            in_copy(k, b).wait()

            @pl.when(k + 1 < NCHUNK)
            def _():
                in_copy(k + 1, 1 - b).start()

            @pl.when(k >= 2)
            def _():
                out_copy(k - 2, b).wait()

            def group_body(g, gcarry):
                m1 = ninf
                m2 = ninf
                a1 = zeros_i
                a2 = zeros_i
                for p in range(P):
                    v = ins[b][p >> 3, p & 7, pl.ds(g * L, L)]
                    gt1 = v > m1
                    u = jnp.minimum(m1, v)
                    ui = jnp.where(gt1, a1, p)
                    gt2 = gt1 | (u > m2)
                    m1 = jnp.maximum(m1, v)
                    a1 = jnp.where(gt1, p, a1)
                    m2 = jnp.maximum(m2, u)
                    a2 = jnp.where(gt2, ui, a2)
                for i in range(P // 8):
                    for q in range(8):
                        outs[b][i, q, pl.ds(g * L, L)] = zeros_i
                tl = lane + g * L
                plsc.store_scatter(
                    outs[b], [a1 >> 3, a1 & 7, tl], ones_i)
                plsc.store_scatter(
                    outs[b], [a2 >> 3, a2 & 7, tl], ones_i)
                return gcarry

            lax.fori_loop(0, GROUPS, group_body, 0)
            out_copy(k, b).start()
        return carry

    lax.fori_loop(0, NCHUNK // 2, pair_body, 0)
    out_copy(NCHUNK - 2, 0).wait()
    out_copy(NCHUNK - 1, 1).wait()


def kernel(score):
    mesh = plsc.VectorSubcoreMesh(
        core_axis_name="c", subcore_axis_name="s",
        num_cores=NC, num_subcores=NS)
    s4 = score.T.reshape(P // 8, 8, N // TB, TB).transpose(0, 2, 1, 3)
    out4 = pl.kernel(
        _topk_mask_body,
        out_type=jax.ShapeDtypeStruct((P // 8, N // TB, 8, TB), jnp.int32),
        mesh=mesh,
        compiler_params=pltpu.CompilerParams(needs_layout_passes=False),
        scratch_types=[
            pltpu.VMEM((P // 8, 8, TB), jnp.float32),
            pltpu.VMEM((P // 8, 8, TB), jnp.float32),
            pltpu.VMEM((P // 8, 8, TB), jnp.int32),
            pltpu.VMEM((P // 8, 8, TB), jnp.int32),
            pltpu.SemaphoreType.DMA,
            pltpu.SemaphoreType.DMA,
            pltpu.SemaphoreType.DMA,
            pltpu.SemaphoreType.DMA,
        ],
    )(s4)
    return out4.transpose(0, 2, 1, 3).reshape(P, N).T


# speedup vs baseline: 13.4515x; 1.0214x over previous
import jax
import jax.numpy as jnp
from jax import lax
from jax.experimental import pallas as pl
from jax.experimental.pallas import tpu as pltpu
from jax.experimental.pallas import tpu_sc as plsc

N = 32768
P = 64
L = 16
NC = 2
NS = 16
NW = NC * NS
TOK_W = N // NW
TB = 128
NCHUNK = TOK_W // TB
GROUPS = TB // L


def _topk_mask_body(in4, out4, in0, in1, ot0, ot1, si0, si1, so0, so1):
    cid = lax.axis_index("c")
    sid = lax.axis_index("s")
    wid = sid * NC + cid
    jbase = wid * NCHUNK

    ins = (in0, in1)
    outs = (ot0, ot1)
    isems = (si0, si1)
    osems = (so0, so1)

    lane = lax.iota(jnp.int32, L)
    zeros_i = jnp.zeros((L,), jnp.int32)
    ones_i = jnp.ones((L,), jnp.int32)
    ninf = jnp.full((L,), float("-inf"), jnp.float32)

    def in_copy(k, b):
        return pltpu.make_async_copy(in4.at[:, jbase + k], ins[b], isems[b])

    def out_copy(k, b):
        return pltpu.make_async_copy(outs[b], out4.at[:, jbase + k], osems[b])

    in_copy(0, 0).start()

    def pair_body(pp, carry):
        for b in range(2):
            k = pp * 2 + b
            in_copy(k, b).wait()

            @pl.when(k + 1 < NCHUNK)
            def _():
                in_copy(k + 1, 1 - b).start()

            @pl.when(k >= 2)
            def _():
                out_copy(k - 2, b).wait()

            def group_body(g, gcarry):
                def octet_body(i, oc):
                    m1, m2, a1, a2 = oc
                    p0 = i * 8
                    for q in range(8):
                        v = ins[b][i, q, pl.ds(g * L, L)]
                        pq = p0 + q
                        gt1 = v > m1
                        u = jnp.minimum(m1, v)
                        ui = jnp.where(gt1, a1, pq)
                        gt2 = gt1 | (u > m2)
                        m1 = jnp.maximum(m1, v)
                        a1 = jnp.where(gt1, pq, a1)
                        m2 = jnp.maximum(m2, u)
                        a2 = jnp.where(gt2, ui, a2)
                        outs[b][i, q, pl.ds(g * L, L)] = zeros_i
                    return m1, m2, a1, a2

                m1, m2, a1, a2 = lax.fori_loop(
                    0, P // 8, octet_body, (ninf, ninf, zeros_i, zeros_i))
                tl = lane + g * L
                plsc.store_scatter(
                    outs[b], [a1 >> 3, a1 & 7, tl], ones_i)
                plsc.store_scatter(
                    outs[b], [a2 >> 3, a2 & 7, tl], ones_i)
                return gcarry

            lax.fori_loop(0, GROUPS, group_body, 0)
            out_copy(k, b).start()
        return carry

    lax.fori_loop(0, NCHUNK // 2, pair_body, 0)
    out_copy(NCHUNK - 2, 0).wait()
    out_copy(NCHUNK - 1, 1).wait()


def kernel(score):
    mesh = plsc.VectorSubcoreMesh(
        core_axis_name="c", subcore_axis_name="s",
        num_cores=NC, num_subcores=NS)
    s4 = score.T.reshape(P // 8, 8, N // TB, TB).transpose(0, 2, 1, 3)
    out4 = pl.kernel(
        _topk_mask_body,
        out_type=jax.ShapeDtypeStruct((P // 8, N // TB, 8, TB), jnp.int32),
        mesh=mesh,
        compiler_params=pltpu.CompilerParams(needs_layout_passes=False),
        scratch_types=[
            pltpu.VMEM((P // 8, 8, TB), jnp.float32),
            pltpu.VMEM((P // 8, 8, TB), jnp.float32),
            pltpu.VMEM((P // 8, 8, TB), jnp.int32),
            pltpu.VMEM((P // 8, 8, TB), jnp.int32),
            pltpu.SemaphoreType.DMA,
            pltpu.SemaphoreType.DMA,
            pltpu.SemaphoreType.DMA,
            pltpu.SemaphoreType.DMA,
        ],
    )(s4)
    return out4.transpose(0, 2, 1, 3).reshape(P, N).T
